# baseline (speedup 1.0000x reference)
import functools

import jax
import jax.numpy as jnp
from jax import lax
from jax.experimental import pallas as pl
from jax.experimental.pallas import tpu as pltpu
from jax.experimental.pallas import tpu_sc as plsc

_NC = 2
_NS = 16


def _proj_kernel(emb_ref, wt_ref, b_ref, p_ref):
    p_ref[...] = (
        jnp.dot(emb_ref[...], wt_ref[...], preferred_element_type=jnp.float32)
        + b_ref[...]
    )


def _compute_table(emb, wt, b2d):
    v = emb.shape[0]
    n = wt.shape[1]
    return pl.pallas_call(
        _proj_kernel,
        out_shape=jax.ShapeDtypeStruct((v, n), jnp.float32),
    )(emb, wt, b2d)


def _sc_gather(p, idx):
    btot = idx.shape[0]
    d = p.shape[1]
    nw = _NC * _NS
    bpw = btot // nw
    chunk = 64
    n_chunks = bpw // chunk
    mesh = plsc.VectorSubcoreMesh(core_axis_name="c", subcore_axis_name="s")

    @functools.partial(
        pl.kernel,
        mesh=mesh,
        out_type=jax.ShapeDtypeStruct((btot, d), jnp.float32),
        scratch_types=[
            pltpu.VMEM((bpw,), jnp.int32),
            pltpu.VMEM((chunk, d), jnp.float32),
            pltpu.SemaphoreType.DMA,
        ],
        compiler_params=pltpu.CompilerParams(use_tc_tiling_on_sc=False),
    )
    def k(p_hbm, idx_hbm, out_hbm, idx_v, rows_v, sem):
        wid = lax.axis_index("s") * _NC + lax.axis_index("c")
        base = wid * bpw
        pltpu.sync_copy(idx_hbm.at[pl.ds(base, bpw)], idx_v)

        def body(c, carry):
            off = c * chunk
            pltpu.async_copy(
                p_hbm.at[idx_v.at[pl.ds(off, chunk)]], rows_v, sem
            ).wait()
            pltpu.sync_copy(rows_v, out_hbm.at[pl.ds(base + off, chunk)])
            return carry

        lax.fori_loop(0, n_chunks, body, 0)

    return k(p, idx)


def kernel(input_seq, emb_table, W, b):
    batch, seq = input_seq.shape
    vocab = W.shape[0]
    idx = input_seq.reshape(-1).astype(jnp.int32)
    p = _compute_table(emb_table, W.T, b.reshape(1, vocab))
    out = _sc_gather(p, idx)
    return out.reshape(batch, seq, vocab)
